# Initial kernel scaffold; baseline (speedup 1.0000x reference)
import jax
import jax.numpy as jnp
from jax.experimental import pallas as pl


def kernel(boxes, scores):
    raise NotImplementedError("write your pallas kernel here")


# speedup vs baseline: 5.8824x; 5.8824x over previous
import jax
import jax.numpy as jnp
from jax.experimental import pallas as pl
from jax.experimental.pallas import tpu as pltpu

_N = 20000
_NPAD = 20480
_ROWS = 160
_PRE = 1000
_PREPAD = 1024
_POST = 100
_NMS_T = 0.6
_SCORE_T = 0.05
_NEG = -1e9


def _nms_body(s_ref, x1_ref, y1_ref, x2_ref, y2_ref, out_ref,
              sw, rm, ss, bsx1, bsy1, bsx2, bsy2,
              bcx1, bcy1, bcx2, bcy2, mm, keep, cnt):
    f32 = jnp.float32

    sw[...] = jnp.where(s_ref[...] > _SCORE_T, s_ref[...], _NEG)
    rm[...] = jnp.reshape(jnp.max(sw[...], axis=1), (1, _ROWS))
    zrow = jnp.zeros((1, _PREPAD), f32)
    bsx1[...] = zrow
    bsy1[...] = zrow
    bsx2[...] = zrow
    bsy2[...] = zrow
    zcol = jnp.zeros((_PREPAD, 1), f32)
    bcx1[...] = zcol
    bcy1[...] = zcol
    bcx2[...] = zcol
    bcy2[...] = zcol
    ss[...] = jnp.full((1, _PREPAD), _NEG, f32)
    keep[...] = jnp.ones((1, _PREPAD), f32)
    cnt[0] = 0

    iota_r = jax.lax.broadcasted_iota(jnp.int32, (1, _ROWS), 1)
    iota_c = jax.lax.broadcasted_iota(jnp.int32, (1, 128), 1)
    iota_s8 = jax.lax.broadcasted_iota(jnp.int32, (8, 1), 0)
    big = jnp.int32(1 << 20)

    def extract(i, _):
        rmv = rm[0:1, :]
        gm = jnp.max(rmv)
        row = jnp.min(jnp.where(rmv == gm, iota_r, big))
        rowv = sw[pl.ds(row, 1), :]
        col = jnp.min(jnp.where(rowv == gm, iota_c, big))
        cmask = iota_c == col
        new_rowv = jnp.where(cmask, -jnp.inf, rowv)
        sw[pl.ds(row, 1), :] = new_rowv
        rm[0:1, :] = jnp.where(iota_r == row, jnp.max(new_rowv), rmv)
        x1v = jnp.sum(jnp.where(cmask, x1_ref[pl.ds(row, 1), :], 0.0))
        y1v = jnp.sum(jnp.where(cmask, y1_ref[pl.ds(row, 1), :], 0.0))
        x2v = jnp.sum(jnp.where(cmask, x2_ref[pl.ds(row, 1), :], 0.0))
        y2v = jnp.sum(jnp.where(cmask, y2_ref[pl.ds(row, 1), :], 0.0))
        blk = pl.multiple_of((i // 128) * 128, 128)
        lmask = iota_c == (i - blk)
        for ref, val in ((ss, gm), (bsx1, x1v), (bsy1, y1v),
                         (bsx2, x2v), (bsy2, y2v)):
            cur = ref[0:1, pl.ds(blk, 128)]
            ref[0:1, pl.ds(blk, 128)] = jnp.where(lmask, val, cur)
        r8 = pl.multiple_of((i // 8) * 8, 8)
        smask = iota_s8 == (i - r8)
        for ref, val in ((bcx1, x1v), (bcy1, y1v), (bcx2, x2v), (bcy2, y2v)):
            cur = ref[pl.ds(r8, 8), :]
            ref[pl.ds(r8, 8), :] = jnp.where(smask, val, cur)
        return 0

    jax.lax.fori_loop(0, _PRE, extract, 0)

    def iou_tile(t, _):
        r0 = t * 8
        xi1 = bcx1[pl.ds(r0, 8), :]
        yi1 = bcy1[pl.ds(r0, 8), :]
        xi2 = bcx2[pl.ds(r0, 8), :]
        yi2 = bcy2[pl.ds(r0, 8), :]
        ai = (xi2 - xi1) * (yi2 - yi1)
        xj1 = bsx1[0:1, :]
        yj1 = bsy1[0:1, :]
        xj2 = bsx2[0:1, :]
        yj2 = bsy2[0:1, :]
        aj = (xj2 - xj1) * (yj2 - yj1)
        w = jnp.maximum(jnp.minimum(xi2, xj2) - jnp.maximum(xi1, xj1), 0.0)
        h = jnp.maximum(jnp.minimum(yi2, yj2) - jnp.maximum(yi1, yj1), 0.0)
        inter = w * h
        iou = inter / (ai + aj - inter + 1e-9)
        mm[pl.ds(r0, 8), :] = (iou > _NMS_T).astype(f32)
        return 0

    jax.lax.fori_loop(0, _PREPAD // 8, iou_tile, 0)

    iota_j = jax.lax.broadcasted_iota(jnp.int32, (1, _PREPAD), 1)

    def emit_row(i, si):
        return jnp.where(
            iota_c == 0, bcx1[i, 0],
            jnp.where(iota_c == 1, bcy1[i, 0],
                      jnp.where(iota_c == 2, bcx2[i, 0],
                                jnp.where(iota_c == 3, bcy2[i, 0], si))))

    def _lane_scalar(ref, i):
        blk = pl.multiple_of((i // 128) * 128, 128)
        v = ref[0:1, pl.ds(blk, 128)]
        return jnp.sum(jnp.where(iota_c == (i - blk), v, 0.0))

    def scan_step(i, _):
        ki = _lane_scalar(keep, i)
        si = _lane_scalar(ss, i)
        mrow = mm[pl.ds(i, 1), :]
        gt = (iota_j > i).astype(f32)
        keep[0:1, :] = keep[0:1, :] * (1.0 - mrow * gt * ki)
        c = cnt[0]

        @pl.when((ki > 0.5) & (si > -5e8) & (c < _POST))
        def _():
            out_ref[pl.ds(c, 1), :] = emit_row(i, si)
            cnt[0] = c + 1

        return 0

    jax.lax.fori_loop(0, _PRE, scan_step, 0)

    @pl.when(cnt[0] < _POST)
    def _fill():
        def fill_step(i, _):
            c = cnt[0]
            ki = _lane_scalar(keep, i)
            si = _lane_scalar(ss, i)

            @pl.when(((ki < 0.5) | (si < -5e8)) & (c < _POST))
            def _():
                out_ref[pl.ds(c, 1), :] = emit_row(i, jnp.float32(_NEG))
                cnt[0] = c + 1

            return 0

        jax.lax.fori_loop(0, _PRE, fill_step, 0)


def _run(boxes, scores, interpret=False):
    f32 = jnp.float32
    pad = _NPAD - _N
    s2d = jnp.pad(scores, (0, pad)).reshape(_ROWS, 128)
    planes = [jnp.pad(boxes[:, k], (0, pad)).reshape(_ROWS, 128)
              for k in range(4)]
    out = pl.pallas_call(
        _nms_body,
        out_shape=jax.ShapeDtypeStruct((_POST, 128), f32),
        scratch_shapes=[
            pltpu.VMEM((_ROWS, 128), f32),
            pltpu.VMEM((1, _ROWS), f32),
            pltpu.VMEM((1, _PREPAD), f32),
            pltpu.VMEM((1, _PREPAD), f32),
            pltpu.VMEM((1, _PREPAD), f32),
            pltpu.VMEM((1, _PREPAD), f32),
            pltpu.VMEM((1, _PREPAD), f32),
            pltpu.VMEM((_PREPAD, 1), f32),
            pltpu.VMEM((_PREPAD, 1), f32),
            pltpu.VMEM((_PREPAD, 1), f32),
            pltpu.VMEM((_PREPAD, 1), f32),
            pltpu.VMEM((_PREPAD, _PREPAD), f32),
            pltpu.VMEM((1, _PREPAD), f32),
            pltpu.SMEM((1,), jnp.int32),
        ],
        interpret=interpret,
    )(s2d, *planes)
    return out[:, :5]


@jax.jit
def _run_compiled(boxes, scores):
    return _run(boxes, scores)


def kernel(boxes, scores):
    return _run_compiled(boxes, scores)
